# Initial kernel scaffold; baseline (speedup 1.0000x reference)
import jax
import jax.numpy as jnp
from jax.experimental import pallas as pl


def kernel(x, w, b):
    raise NotImplementedError("write your pallas kernel here")


# speedup vs baseline: 5.0876x; 5.0876x over previous
import jax
import jax.numpy as jnp
from jax.experimental import pallas as pl
from jax.experimental.pallas import tpu as pltpu


def _fc_sigmoid_kernel(x_ref, wt_ref, b_ref, o_ref):
    xb = x_ref[...].astype(jnp.bfloat16)
    acc = jnp.dot(xb, wt_ref[...], preferred_element_type=jnp.float32)
    o_ref[...] = jax.nn.sigmoid(acc + b_ref[...])


def kernel(x, w, b):
    B, In = x.shape
    Out, In_w = w.shape
    assert In == In_w and b.shape == (Out,)

    wt = w.T.astype(jnp.bfloat16)
    b2 = b.reshape(1, Out)

    tm = min(512, B)
    assert B % tm == 0
    out = pl.pallas_call(
        _fc_sigmoid_kernel,
        out_shape=jax.ShapeDtypeStruct((B, Out), jnp.float32),
        grid=(B // tm,),
        in_specs=[
            pl.BlockSpec((tm, In), lambda i: (i, 0)),
            pl.BlockSpec((In, Out), lambda i: (0, 0)),
            pl.BlockSpec((1, Out), lambda i: (0, 0)),
        ],
        out_specs=pl.BlockSpec((tm, Out), lambda i: (i, 0)),
        compiler_params=pltpu.CompilerParams(
            dimension_semantics=("parallel",)),
    )(x, wt, b2)
    return out


# speedup vs baseline: 5.9798x; 1.1754x over previous
import jax
import jax.numpy as jnp
from jax.experimental import pallas as pl
from jax.experimental.pallas import tpu as pltpu


def _fc_sigmoid_kernel(x_ref, w_ref, b_ref, o_ref):
    xb = x_ref[...].astype(jnp.bfloat16)
    wb = w_ref[...].astype(jnp.bfloat16)
    acc = jax.lax.dot_general(
        xb, wb, (((1,), (1,)), ((), ())),
        preferred_element_type=jnp.float32)
    o_ref[...] = jax.nn.sigmoid(acc + b_ref[...])


def kernel(x, w, b):
    B, In = x.shape
    Out, In_w = w.shape
    assert In == In_w and b.shape == (Out,)

    b2 = b.reshape(1, Out)

    tm = min(512, B)
    assert B % tm == 0
    out = pl.pallas_call(
        _fc_sigmoid_kernel,
        out_shape=jax.ShapeDtypeStruct((B, Out), jnp.float32),
        grid=(B // tm,),
        in_specs=[
            pl.BlockSpec((tm, In), lambda i: (i, 0)),
            pl.BlockSpec((Out, In), lambda i: (0, 0)),
            pl.BlockSpec((1, Out), lambda i: (0, 0)),
        ],
        out_specs=pl.BlockSpec((tm, Out), lambda i: (i, 0)),
        compiler_params=pltpu.CompilerParams(
            dimension_semantics=("parallel",)),
    )(x, w, b2)
    return out


# speedup vs baseline: 6.7120x; 1.1224x over previous
import jax
import jax.numpy as jnp
from jax.experimental import pallas as pl
from jax.experimental.pallas import tpu as pltpu


def _fc_sigmoid_kernel(x_ref, w_ref, b_ref, o_ref):
    xb = x_ref[...].astype(jnp.bfloat16)
    wb = w_ref[...].astype(jnp.bfloat16)
    acc = jax.lax.dot_general(
        xb, wb, (((1,), (1,)), ((), ())),
        preferred_element_type=jnp.float32)
    o_ref[...] = jax.nn.sigmoid(acc + b_ref[...])


def kernel(x, w, b):
    B, In = x.shape
    Out, In_w = w.shape
    assert In == In_w and b.shape == (Out,)

    b2 = b.reshape(1, Out)

    tm = min(1024, B)
    assert B % tm == 0
    out = pl.pallas_call(
        _fc_sigmoid_kernel,
        out_shape=jax.ShapeDtypeStruct((B, Out), jnp.float32),
        grid=(B // tm,),
        in_specs=[
            pl.BlockSpec((tm, In), lambda i: (i, 0)),
            pl.BlockSpec((Out, In), lambda i: (0, 0)),
            pl.BlockSpec((1, Out), lambda i: (0, 0)),
        ],
        out_specs=pl.BlockSpec((tm, Out), lambda i: (i, 0)),
        compiler_params=pltpu.CompilerParams(
            dimension_semantics=("parallel",)),
    )(x, w, b2)
    return out
